# Initial kernel scaffold; baseline (speedup 1.0000x reference)
import jax
import jax.numpy as jnp
from jax.experimental import pallas as pl


def kernel(scores, labels, feat, pick_val, feat_bank, bank_confidence_transport, bank_confidence):
    raise NotImplementedError("write your pallas kernel here")


# speedup vs baseline: 41.1688x; 41.1688x over previous
import functools

import jax
import jax.numpy as jnp
from jax import lax
from jax.experimental import pallas as pl
from jax.experimental.pallas import tpu as pltpu
from jax.experimental.pallas import tpu_sc as plsc

_BS = 1024
_NCLS = 1000
_NPAD = 1024
_DIM = 512
_SMAX = 32
_ROW = _DIM * _SMAX
_NEG = -(2 ** 31)


def _judge_body(scores_ref, labels_ref, bc_ref, pick_ref, bct_ref,
                wu_ref, bcnew_ref, bctnew_ref):
    s = scores_ref[...]
    lab = labels_ref[...]
    bc = bc_ref[...]

    m = jnp.max(s, axis=1, keepdims=True)
    denom = jnp.sum(jnp.exp(s - m), axis=1, keepdims=True)
    pred_val = 1.0 / denom
    col = lax.broadcasted_iota(jnp.int32, (_BS, _NPAD), 1)
    rowi = lax.broadcasted_iota(jnp.int32, (_BS, _NPAD), 0)
    is_max = s == m
    pred_pos = jnp.min(jnp.where(is_max, col, jnp.int32(2 ** 30)),
                       axis=1, keepdims=True)

    labmask = col == lab
    lbl_conf = jnp.sum(jnp.where(labmask, bc, 0.0), axis=1, keepdims=True)

    correct = pred_pos == lab
    u = correct & ((pred_val - lbl_conf) > 0.1)
    f = correct & ((lbl_conf - pred_val) > 0.1) & (lbl_conf != 0.0)

    neg1 = jnp.int32(-1)
    w_u = jnp.max(jnp.where(labmask & u, rowi, neg1), axis=0, keepdims=True)
    w_uf = jnp.max(jnp.where(labmask & (u | f), rowi, neg1),
                   axis=0, keepdims=True)

    pv_at = jnp.max(jnp.where(labmask & (rowi == w_uf), pred_val, -jnp.inf),
                    axis=0, keepdims=True)
    u_last = (w_uf == w_u) & (w_uf >= 0)
    bcnew_ref[...] = jnp.where(
        w_uf < 0, bc, jnp.where(u_last, pv_at, bc - jnp.float32(0.1 / _BS)))

    oh = ((rowi == w_u) & (w_u >= 0)).astype(jnp.float32)
    sel = lax.dot_general(oh, pick_ref[...],
                          dimension_numbers=(((0,), (0,)), ((), ())),
                          preferred_element_type=jnp.float32)
    cnt = lax.dot_general(oh, jnp.ones((_BS, 1), jnp.float32),
                          dimension_numbers=(((0,), (0,)), ((), ())),
                          preferred_element_type=jnp.float32)
    has_u = cnt[: _NCLS] > 0.5
    bctnew_ref[...] = jnp.where(has_u, sel[: _NCLS], bct_ref[...])

    wu_ref[...] = w_u


_judge_call = pl.pallas_call(
    _judge_body,
    out_shape=(
        jax.ShapeDtypeStruct((1, _NPAD), jnp.int32),
        jax.ShapeDtypeStruct((1, _NPAD), jnp.float32),
        jax.ShapeDtypeStruct((_NCLS, _SMAX), jnp.float32),
    ),
)


def _sc_body(feat_hbm, bank_hbm, wu_hbm, out_hbm, wu_v, buf):
    wid = lax.axis_index("s") * 2 + lax.axis_index("c")
    base = wid * 32
    pltpu.sync_copy(wu_hbm.at[pl.ds(base, 32)], wu_v)
    for j in range(32):
        c = base + j
        vec = wu_v[pl.ds((j // 16) * 16, 16)]
        wu_c = vec[j % 16]

        @pl.when(c < _NCLS)
        def _copy_row(c=c, wu_c=wu_c):
            use_feat = wu_c >= 0
            src = jnp.where(use_feat, wu_c, 0)

            @pl.when(use_feat)
            def _():
                pltpu.sync_copy(feat_hbm.at[pl.ds(src, 1)], buf)

            @pl.when(jnp.logical_not(use_feat))
            def _():
                pltpu.sync_copy(bank_hbm.at[pl.ds(c, 1)], buf)

            pltpu.sync_copy(buf, out_hbm.at[pl.ds(c, 1)])


def _sc_call(feat2, bank2, wu1):
    call = pl.kernel(
        _sc_body,
        out_type=jax.ShapeDtypeStruct((_NCLS, _ROW), jnp.float32),
        mesh=plsc.VectorSubcoreMesh(core_axis_name="c", subcore_axis_name="s"),
        scratch_types=[
            pltpu.VMEM((32,), jnp.int32),
            pltpu.VMEM((1, _ROW), jnp.float32),
        ],
    )
    return call(feat2, bank2, wu1)


def kernel(scores, labels, feat, pick_val, feat_bank,
           bank_confidence_transport, bank_confidence):
    scores_p = jnp.pad(scores, ((0, 0), (0, _NPAD - _NCLS)),
                       constant_values=-1e30)
    labels2 = labels.astype(jnp.int32).reshape(_BS, 1)
    bc_p = jnp.pad(bank_confidence, (0, _NPAD - _NCLS)).reshape(1, _NPAD)

    w_u, bc_new, bct_new = _judge_call(
        scores_p, labels2, bc_p, pick_val, bank_confidence_transport)

    feat2 = feat.reshape(_BS, _ROW)
    bank2 = feat_bank.reshape(_NCLS, _ROW)
    out2 = _sc_call(feat2, bank2, w_u.reshape(_NPAD))

    return (out2.reshape(_NCLS, _DIM, _SMAX),
            bct_new,
            bc_new.reshape(_NPAD)[: _NCLS])
